# baseline (speedup 1.0000x reference)
import functools

import jax
import jax.numpy as jnp
from jax import lax
from jax.experimental import pallas as pl
from jax.experimental.pallas import tpu as pltpu
from jax.experimental.pallas import tpu_sc as plsc

N_NODES = 10000
N_EDGES = 320000
D = 128
N_GRAPHS = 64

_NC, _NS = 2, 16
_NW = _NC * _NS
_CHUNK = 128
_K = 80
_KH = _K // 2
_NE_PAD = _NW * _K * _CHUNK
_R_AGG = 10112
_R_DUMP = _R_AGG - N_NODES
_ZROWS = _R_AGG // _NS

_DOT = dict(preferred_element_type=jnp.float32, precision=lax.Precision.DEFAULT)
_DOT_X = dict(preferred_element_type=jnp.float32, precision=lax.Precision.HIGHEST)


def _prelu(v, a):
    return jnp.where(v > 0, v, a * v)


def _pre_body(x_ref, w1_ref, b1_ref, a1_ref, w2_ref, b2_ref, a2_ref,
              gamma_ref, beta_ref, wm_ref, bm_ref, h_ref, g_ref):
    h1 = _prelu(jnp.dot(x_ref[...], w1_ref[...], **_DOT) + b1_ref[...], a1_ref[...])
    h2 = _prelu(jnp.dot(h1, w2_ref[...], **_DOT) + b2_ref[...], a2_ref[...])
    mu = jnp.mean(h2, axis=0, keepdims=True)
    var = jnp.mean((h2 - mu) ** 2, axis=0, keepdims=True)
    h = (h2 - mu) / jnp.sqrt(var + 1e-5) * gamma_ref[...] + beta_ref[...]
    h_ref[...] = h
    g_ref[...] = jnp.dot(h, wm_ref[...], **_DOT) + bm_ref[...]


def _layer_body(aggr_ref, h_ref, a_ref, wm_ref, bm_ref, ho_ref, go_ref):
    aggr = aggr_ref[0, :N_NODES, :] + aggr_ref[1, :N_NODES, :]
    h = _prelu(aggr + h_ref[...], a_ref[...])
    ho_ref[...] = h
    go_ref[...] = jnp.dot(h, wm_ref[...], **_DOT) + bm_ref[...]


def _last_body(aggr_ref, h_ref, a_ref, ho_ref):
    aggr = aggr_ref[0, :N_NODES, :] + aggr_ref[1, :N_NODES, :]
    ho_ref[...] = _prelu(aggr + h_ref[...], a_ref[...])


def _pool_body(h0_ref, h1_ref, h2_ref, h3_ref, h4_ref, bi_ref,
               wf1_ref, bf1_ref, wf2_ref, bf2_ref, out_ref):
    gid = lax.broadcasted_iota(jnp.int32, (N_GRAPHS, N_NODES), 0)
    onehot = jnp.where(gid == bi_ref[...], 1.0, 0.0).astype(jnp.float32)
    parts = [jnp.dot(onehot, h[...], **_DOT_X)
             for h in (h0_ref, h1_ref, h2_ref, h3_ref, h4_ref)]
    pooled = jnp.concatenate(parts, axis=1)
    z = jnp.maximum(jnp.dot(pooled, wf1_ref[...], **_DOT) + bf1_ref[...], 0.0)
    out_ref[...] = jnp.dot(z, wf2_ref[...], **_DOT) + bf2_ref[...]


_f32 = jnp.float32
_nd = jax.ShapeDtypeStruct((N_NODES, D), _f32)

_pre_call = pl.pallas_call(_pre_body, out_shape=(_nd, _nd))
_layer_call = pl.pallas_call(_layer_body, out_shape=(_nd, _nd))
_last_call = pl.pallas_call(_last_body, out_shape=_nd)
_pool_call = pl.pallas_call(
    _pool_body, out_shape=jax.ShapeDtypeStruct((N_GRAPHS, 1), _f32))


def _sc_body(g_hbm, src_hbm, dst_hbm, zeros_hbm, out_hbm,
             src_v, dst_v, rows0_v, rows1_v, aggr_s,
             semg0, semg1, sems0, sems1):
    cid = lax.axis_index("c")
    sid = lax.axis_index("s")
    wid = sid * _NC + cid
    pltpu.sync_copy(zeros_hbm, aggr_s.at[pl.ds(sid * _ZROWS, _ZROWS)])
    plsc.subcore_barrier()

    for half in range(2):
        base = half * _KH
        pltpu.sync_copy(src_hbm.at[wid, pl.ds(base, _KH)], src_v)
        pltpu.sync_copy(dst_hbm.at[wid, pl.ds(base, _KH)], dst_v)
        pltpu.async_copy(g_hbm.at[src_v.at[0]], rows0_v, semg0)
        pltpu.async_copy(g_hbm.at[src_v.at[1]], rows1_v, semg1)

        def pair(jj, carry):
            j0 = jj * 2
            j1 = j0 + 1
            pltpu.make_async_copy(g_hbm.at[src_v.at[j0]], rows0_v, semg0).wait()
            pltpu.async_copy(rows0_v, aggr_s.at[dst_v.at[j0]], sems0, add=True)
            pltpu.make_async_copy(g_hbm.at[src_v.at[j1]], rows1_v, semg1).wait()
            pltpu.async_copy(rows1_v, aggr_s.at[dst_v.at[j1]], sems1, add=True)

            @pl.when(jj + 1 < _KH // 2)
            def _():
                pltpu.make_async_copy(
                    rows0_v, aggr_s.at[dst_v.at[j0]], sems0).wait()
                pltpu.async_copy(g_hbm.at[src_v.at[j0 + 2]], rows0_v, semg0)
                pltpu.make_async_copy(
                    rows1_v, aggr_s.at[dst_v.at[j1]], sems1).wait()
                pltpu.async_copy(g_hbm.at[src_v.at[j1 + 2]], rows1_v, semg1)
            return carry

        lax.fori_loop(0, _KH // 2, pair, 0)
        pltpu.make_async_copy(rows0_v, aggr_s.at[dst_v.at[_KH - 2]], sems0).wait()
        pltpu.make_async_copy(rows1_v, aggr_s.at[dst_v.at[_KH - 1]], sems1).wait()
    plsc.subcore_barrier()
    pltpu.sync_copy(aggr_s.at[pl.ds(sid * _ZROWS, _ZROWS)],
                    out_hbm.at[cid, pl.ds(sid * _ZROWS, _ZROWS)])


@functools.cache
def _sc_edge_aggregate_call():
    return pl.kernel(
        _sc_body,
        out_type=jax.ShapeDtypeStruct((_NC, _R_AGG, D), _f32),
        mesh=plsc.VectorSubcoreMesh(core_axis_name="c", subcore_axis_name="s"),
        scratch_types=[
            pltpu.VMEM((_KH, _CHUNK), jnp.int32),
            pltpu.VMEM((_KH, _CHUNK), jnp.int32),
            pltpu.VMEM((_CHUNK, D), _f32),
            pltpu.VMEM((_CHUNK, D), _f32),
            pltpu.VMEM_SHARED((_R_AGG, D), _f32),
            pltpu.SemaphoreType.DMA,
            pltpu.SemaphoreType.DMA,
            pltpu.SemaphoreType.DMA,
            pltpu.SemaphoreType.DMA,
        ],
    )


def kernel(x, edge_index, batchidxs, W1, b1, a1, W2, b2, a2, gamma, beta, act_a,
           Wm0, bm0, Wm1, bm1, Wm2, bm2, Wm3, bm3, Wf1, bf1, Wf2, bf2):
    r1 = lambda v: v.reshape(1, -1)
    src = edge_index[0].astype(jnp.int32)
    dst = edge_index[1].astype(jnp.int32)
    npad = _NE_PAD - N_EDGES
    pad_src = (jnp.arange(npad, dtype=jnp.int32) * 131) % N_NODES
    pad_dst = N_NODES + (jnp.arange(npad, dtype=jnp.int32) % _R_DUMP)
    src3 = jnp.concatenate([src, pad_src]).reshape(_NW, _K, _CHUNK)
    dst3 = jnp.concatenate([dst, pad_dst]).reshape(_NW, _K, _CHUNK)
    zeros = jnp.zeros((_ZROWS, D), _f32)

    h, g = _pre_call(x, W1, r1(b1), r1(a1), W2, r1(b2), r1(a2),
                     r1(gamma), r1(beta), Wm0, r1(bm0))
    hs = [h]
    next_w = ((Wm1, bm1), (Wm2, bm2), (Wm3, bm3))
    sc_aggregate = _sc_edge_aggregate_call()
    for i in range(4):
        aggr = sc_aggregate(g, src3, dst3, zeros)
        if i < 3:
            wm, bm = next_w[i]
            h, g = _layer_call(aggr, h, r1(act_a), wm, r1(bm))
        else:
            h = _last_call(aggr, h, r1(act_a))
        hs.append(h)
    return _pool_call(*hs, batchidxs.reshape(1, -1).astype(jnp.int32),
                      Wf1, r1(bf1), Wf2, r1(bf2))


# speedup vs baseline: 1.2686x; 1.2686x over previous
import functools

import jax
import jax.numpy as jnp
from jax import lax
from jax.experimental import pallas as pl
from jax.experimental.pallas import tpu as pltpu
from jax.experimental.pallas import tpu_sc as plsc

N_NODES = 10000
N_EDGES = 320000
D = 128
N_GRAPHS = 64

_NC, _NS = 2, 16
_NW = _NC * _NS
_CHUNK = 128
_K = 80
_KH = _K // 2
_NE_PAD = _NW * _K * _CHUNK
_R_AGG = 10112
_R_DUMP = _R_AGG - N_NODES
_ZROWS = _R_AGG // _NS

_DOT = dict(preferred_element_type=jnp.float32, precision=lax.Precision.DEFAULT)
_DOT_X = dict(preferred_element_type=jnp.float32, precision=lax.Precision.HIGHEST)


def _prelu(v, a):
    return jnp.where(v > 0, v, a * v)


def _pre_body(x_ref, w1_ref, b1_ref, a1_ref, w2_ref, b2_ref, a2_ref,
              gamma_ref, beta_ref, wm_ref, bm_ref, h_ref, g_ref):
    h1 = _prelu(jnp.dot(x_ref[...], w1_ref[...], **_DOT) + b1_ref[...], a1_ref[...])
    h2 = _prelu(jnp.dot(h1, w2_ref[...], **_DOT) + b2_ref[...], a2_ref[...])
    mu = jnp.mean(h2, axis=0, keepdims=True)
    var = jnp.mean((h2 - mu) ** 2, axis=0, keepdims=True)
    h = (h2 - mu) / jnp.sqrt(var + 1e-5) * gamma_ref[...] + beta_ref[...]
    h_ref[...] = h
    g_ref[...] = jnp.dot(h, wm_ref[...], **_DOT) + bm_ref[...]


def _layer_body(aggr_ref, h_ref, a_ref, wm_ref, bm_ref, ho_ref, go_ref):
    aggr = aggr_ref[0, :N_NODES, :] + aggr_ref[1, :N_NODES, :]
    h = _prelu(aggr + h_ref[...], a_ref[...])
    ho_ref[...] = h
    go_ref[...] = jnp.dot(h, wm_ref[...], **_DOT) + bm_ref[...]


def _last_body(aggr_ref, h_ref, a_ref, ho_ref):
    aggr = aggr_ref[0, :N_NODES, :] + aggr_ref[1, :N_NODES, :]
    ho_ref[...] = _prelu(aggr + h_ref[...], a_ref[...])


def _pool_body(h0_ref, h1_ref, h2_ref, h3_ref, h4_ref, bi_ref,
               wf1_ref, bf1_ref, wf2_ref, bf2_ref, out_ref):
    gid = lax.broadcasted_iota(jnp.int32, (N_GRAPHS, N_NODES), 0)
    onehot = jnp.where(gid == bi_ref[...], 1.0, 0.0).astype(jnp.float32)
    parts = [jnp.dot(onehot, h[...], **_DOT_X)
             for h in (h0_ref, h1_ref, h2_ref, h3_ref, h4_ref)]
    pooled = jnp.concatenate(parts, axis=1)
    z = jnp.maximum(jnp.dot(pooled, wf1_ref[...], **_DOT) + bf1_ref[...], 0.0)
    out_ref[...] = jnp.dot(z, wf2_ref[...], **_DOT) + bf2_ref[...]


_f32 = jnp.float32
_nd = jax.ShapeDtypeStruct((N_NODES, D), _f32)

_pre_call = pl.pallas_call(_pre_body, out_shape=(_nd, _nd))
_layer_call = pl.pallas_call(_layer_body, out_shape=(_nd, _nd))
_last_call = pl.pallas_call(_last_body, out_shape=_nd)
_pool_call = pl.pallas_call(
    _pool_body, out_shape=jax.ShapeDtypeStruct((N_GRAPHS, 1), _f32))


def _sc_body(g_hbm, src_hbm, dst_hbm, zeros_hbm, out_hbm,
             src_v, dst_v, rows0_v, rows1_v, aggr_s, semg0, semg1):
    cid = lax.axis_index("c")
    sid = lax.axis_index("s")
    wid = sid * _NC + cid
    pltpu.sync_copy(zeros_hbm, aggr_s.at[pl.ds(sid * _ZROWS, _ZROWS)])
    plsc.subcore_barrier()

    for half in range(2):
        base = half * _KH
        pltpu.sync_copy(src_hbm.at[wid, pl.ds(base, _KH)], src_v)
        pltpu.sync_copy(dst_hbm.at[wid, pl.ds(base, _KH)], dst_v)
        pltpu.async_copy(g_hbm.at[src_v.at[0]], rows0_v, semg0)

        def pair(jj, carry):
            j0 = jj * 2
            j1 = j0 + 1
            pltpu.async_copy(g_hbm.at[src_v.at[j1]], rows1_v, semg1)
            pltpu.make_async_copy(g_hbm.at[src_v.at[j0]], rows0_v, semg0).wait()
            pltpu.sync_copy(rows0_v, aggr_s.at[dst_v.at[j0]], add=True)

            @pl.when(jj + 1 < _KH // 2)
            def _():
                pltpu.async_copy(g_hbm.at[src_v.at[j0 + 2]], rows0_v, semg0)

            pltpu.make_async_copy(g_hbm.at[src_v.at[j1]], rows1_v, semg1).wait()
            pltpu.sync_copy(rows1_v, aggr_s.at[dst_v.at[j1]], add=True)
            return carry

        lax.fori_loop(0, _KH // 2, pair, 0)
    plsc.subcore_barrier()
    pltpu.sync_copy(aggr_s.at[pl.ds(sid * _ZROWS, _ZROWS)],
                    out_hbm.at[cid, pl.ds(sid * _ZROWS, _ZROWS)])


@functools.cache
def _sc_edge_aggregate_call():
    return pl.kernel(
        _sc_body,
        out_type=jax.ShapeDtypeStruct((_NC, _R_AGG, D), _f32),
        mesh=plsc.VectorSubcoreMesh(core_axis_name="c", subcore_axis_name="s"),
        scratch_types=[
            pltpu.VMEM((_KH, _CHUNK), jnp.int32),
            pltpu.VMEM((_KH, _CHUNK), jnp.int32),
            pltpu.VMEM((_CHUNK, D), _f32),
            pltpu.VMEM((_CHUNK, D), _f32),
            pltpu.VMEM_SHARED((_R_AGG, D), _f32),
            pltpu.SemaphoreType.DMA,
            pltpu.SemaphoreType.DMA,
        ],
    )


def kernel(x, edge_index, batchidxs, W1, b1, a1, W2, b2, a2, gamma, beta, act_a,
           Wm0, bm0, Wm1, bm1, Wm2, bm2, Wm3, bm3, Wf1, bf1, Wf2, bf2):
    r1 = lambda v: v.reshape(1, -1)
    src = edge_index[0].astype(jnp.int32)
    dst = edge_index[1].astype(jnp.int32)
    npad = _NE_PAD - N_EDGES
    pad_src = (jnp.arange(npad, dtype=jnp.int32) * 131) % N_NODES
    pad_dst = N_NODES + (jnp.arange(npad, dtype=jnp.int32) % _R_DUMP)
    src3 = jnp.concatenate([src, pad_src]).reshape(_NW, _K, _CHUNK)
    dst3 = jnp.concatenate([dst, pad_dst]).reshape(_NW, _K, _CHUNK)
    zeros = jnp.zeros((_ZROWS, D), _f32)

    h, g = _pre_call(x, W1, r1(b1), r1(a1), W2, r1(b2), r1(a2),
                     r1(gamma), r1(beta), Wm0, r1(bm0))
    hs = [h]
    next_w = ((Wm1, bm1), (Wm2, bm2), (Wm3, bm3))
    sc_aggregate = _sc_edge_aggregate_call()
    for i in range(4):
        aggr = sc_aggregate(g, src3, dst3, zeros)
        if i < 3:
            wm, bm = next_w[i]
            h, g = _layer_call(aggr, h, r1(act_a), wm, r1(bm))
        else:
            h = _last_call(aggr, h, r1(act_a))
        hs.append(h)
    return _pool_call(*hs, batchidxs.reshape(1, -1).astype(jnp.int32),
                      Wf1, r1(bf1), Wf2, r1(bf2))
